# baseline (speedup 1.0000x reference)
import functools

import jax
import jax.numpy as jnp
from jax import lax
from jax.experimental import pallas as pl
from jax.experimental.pallas import tpu as pltpu
from jax.experimental.pallas import tpu_sc as plsc

B = 16384
D = 32
V = 100001
RBLK = 4096
Q = 7 * RBLK
NVBLK = -(-V // RBLK)

_info = plsc.get_sparse_core_info()
_NC = _info.num_cores
_NS = _info.num_subcores
_NW = _NC * _NS
_BPW = B // _NW

_mesh = plsc.VectorSubcoreMesh(core_axis_name="c", subcore_axis_name="s")


def _pack_body(t0, t1, t2, t3, w1, b1, w2, b2, o_ref):
    outs = []
    for t in (t0, t1, t2, t3):
        x = t[...]
        h = jnp.maximum(
            jnp.dot(x, w1[...], preferred_element_type=jnp.float32) + b1[...],
            0.0,
        )
        outs.append(
            jnp.dot(h, w2[...], preferred_element_type=jnp.float32) + b2[...]
        )
    o_ref[...] = jnp.concatenate(outs, axis=1)


def _table_mlp_packed(table, W1, b1, W2, b2):
    nq = Q // RBLK

    def tmap(c):
        return lambda i: (jnp.minimum(nq * c + i, NVBLK - 1), 0)

    return pl.pallas_call(
        _pack_body,
        grid=(nq,),
        in_specs=[
            pl.BlockSpec((RBLK, D), tmap(0)),
            pl.BlockSpec((RBLK, D), tmap(1)),
            pl.BlockSpec((RBLK, D), tmap(2)),
            pl.BlockSpec((RBLK, D), tmap(3)),
            pl.BlockSpec(W1.shape, lambda i: (0, 0)),
            pl.BlockSpec((1, W1.shape[1]), lambda i: (0, 0)),
            pl.BlockSpec(W2.shape, lambda i: (0, 0)),
            pl.BlockSpec((1, W2.shape[1]), lambda i: (0, 0)),
        ],
        out_specs=pl.BlockSpec((RBLK, 4 * D), lambda i: (i, 0)),
        out_shape=jax.ShapeDtypeStruct((Q, 4 * D), jnp.float32),
    )(table, table, table, table, W1, b1.reshape(1, -1), W2, b2.reshape(1, -1))


@functools.partial(
    pl.kernel,
    mesh=_mesh,
    out_type=jax.ShapeDtypeStruct((B, 4 * D), jnp.float32),
    scratch_types=[
        pltpu.VMEM((_BPW,), jnp.int32),
        pltpu.VMEM((_BPW,), jnp.int32),
        pltpu.VMEM((_BPW, 4 * D), jnp.float32),
        pltpu.SemaphoreType.DMA,
    ],
)
def _sc_gather(packed_hbm, idx_hbm, out_hbm, idx_v, j_v, rows_v, sem):
    wid = lax.axis_index("s") * _NC + lax.axis_index("c")
    base = wid * _BPW
    pltpu.sync_copy(idx_hbm.at[pl.ds(base, _BPW)], idx_v)
    for k in range(_BPW // 16):
        sl = pl.ds(k * 16, 16)
        j_v[sl] = lax.rem(idx_v[sl], Q)
    pltpu.async_copy(packed_hbm.at[j_v], rows_v, sem).wait()
    pltpu.sync_copy(rows_v, out_hbm.at[pl.ds(base, _BPW)])


def _select_body(g_ref, uid_ref, o_ref):
    slot = uid_ref[...] // Q
    g = g_ref[...]
    o_ref[...] = jnp.where(
        slot < 2,
        jnp.where(slot == 0, g[:, 0:D], g[:, D:2 * D]),
        jnp.where(slot == 2, g[:, 2 * D:3 * D], g[:, 3 * D:4 * D]),
    )


def _select(gathered, user_id):
    blk = 4096
    return pl.pallas_call(
        _select_body,
        grid=(B // blk,),
        in_specs=[
            pl.BlockSpec((blk, 4 * D), lambda i: (i, 0)),
            pl.BlockSpec((blk, 1), lambda i: (i, 0)),
        ],
        out_specs=pl.BlockSpec((blk, D), lambda i: (i, 0)),
        out_shape=jax.ShapeDtypeStruct((B, D), jnp.float32),
    )(gathered, user_id.reshape(B, 1))


def kernel(user_id, table, W1, b1, W2, b2):
    uid = user_id.astype(jnp.int32)
    packed = _table_mlp_packed(table, W1, b1, W2, b2)
    gathered = _sc_gather(packed, uid)
    return _select(gathered, uid)
